# baseline (speedup 1.0000x reference)
import functools

import jax
import jax.numpy as jnp
from jax import lax
from jax.experimental import pallas as pl
from jax.experimental.pallas import tpu as pltpu
from jax.experimental.pallas import tpu_sc as plsc

N = 50000
E = 1600000
NTILES = 16
NP = 51200
NROW = NP // 128
NS = NP // NTILES
NVR = NS // 16
RPT = NROW // NTILES
ZR = 5
EPT = E // NTILES
CH = 800
NCH = EPT // CH
CHV = CH // 16
NBANK = 3
PUB = 80
NPUB = NROW // PUB


def _rsqrt16(x):
    xi = plsc.bitcast(x, jnp.int32)
    yi = jnp.int32(0x5F3759DF) - (xi >> 1)
    y = plsc.bitcast(yi, jnp.float32)
    for _ in range(3):
        y = y * (1.5 - 0.5 * x * y * y)
    return y


def _sc_body(ei_hbm, zeros_hbm, a_out,
             t_loc, u_loc, sb0, db0, sb1, db1, sb2, db2,
             ubuf, zbuf, idxq, ebuf_d, ebuf_a, ebuf_t, spm_t, spm_u,
             sem0, sem1, sem2, semz):
    wid = lax.axis_index("s")
    sl = pl.ds(wid * NS, NS)
    rsl = pl.ds(wid * RPT, RPT)
    ebase = wid * EPT
    zero16 = jnp.zeros((16,), jnp.float32)
    ones16 = jnp.ones((16,), jnp.float32)
    iota16 = lax.iota(jnp.int32, 16)

    def _z(k, _):
        r = k // 8
        c = lax.rem(k, 8)
        zbuf[r, pl.ds(c * 16, 16)] = zero16
        return 0
    lax.fori_loop(0, ZR * 8, _z, 0)
    for q in range(NPUB):
        def _iq(k, _):
            idxq[q, pl.ds(k * 16, 16)] = iota16 + (q * PUB + k * 16)
            return 0
        lax.fori_loop(0, PUB // 16, _iq, 0)

    def _reset_spm_u_rows():
        cps = [pltpu.async_copy(zbuf, spm_u.at[pl.ds(wid * RPT + g * ZR, ZR)], sem0)
               for g in range(RPT // ZR)]
        for cp in cps:
            cp.wait()

    _reset_spm_u_rows()
    zcp = pltpu.async_copy(zeros_hbm, u_loc, semz)
    zcp.wait()
    plsc.subcore_barrier()

    banks = ((sb0, db0, sem0), (sb1, db1, sem1), (sb2, db2, sem2))

    def _edge_loop(deg_only, process16):
        def _fetch(c, bank):
            sb, db, sem = banks[bank]
            if not deg_only:
                pltpu.async_copy(ei_hbm.at[pl.ds(ebase + c * CH, CH)], sb, sem)
            pltpu.async_copy(ei_hbm.at[pl.ds(E + ebase + c * CH, CH)], db, sem)

        for b in range(NBANK - 1):
            _fetch(b, b)

        def _chunk(c, _):
            def _go(bank):
                sb, db, sem = banks[bank]
                for _ in range(1 if deg_only else 2):
                    pltpu.make_async_copy(
                        ei_hbm.at[pl.ds(0, CH)], db, sem).wait()
                @pl.when(c + (NBANK - 1) < NCH)
                def _():
                    _fetch(c + (NBANK - 1), (bank + NBANK - 1) % NBANK)
                @plsc.parallel_loop(0, CHV, unroll=10)
                def _(i):
                    process16(sb[pl.ds(i * 16, 16)], db[pl.ds(i * 16, 16)])
            for b in range(NBANK):
                @pl.when(lax.rem(c, NBANK) == b)
                def _(b=b):
                    _go(b)
            return 0
        lax.fori_loop(0, NCH, _chunk, 0)

    def _publish():
        cps = [pltpu.async_copy(u_loc.at[pl.ds(q * PUB, PUB)],
                                spm_u.at[idxq.at[q]], sem0, add=True)
               for q in range(NPUB)]
        for cp in cps:
            cp.wait()
        return pltpu.async_copy(zeros_hbm, u_loc, semz)

    def _deg16(s, d):
        del s
        plsc.addupdate_scatter(u_loc, [d >> 7, d & 127], ones16)
    _edge_loop(True, _deg16)
    zcp = _publish()
    plsc.subcore_barrier()

    pltpu.sync_copy(spm_u.at[rsl], ubuf)
    _reset_spm_u_rows()
    @plsc.parallel_loop(0, NVR, unroll=4)
    def _(k):
        r = k // 8
        c = lax.rem(k, 8)
        acc = ubuf[r, pl.ds(c * 16, 16)]
        dv = _rsqrt16(acc + 1.0)
        ebuf_d[pl.ds(k * 16, 16)] = dv
    pltpu.sync_copy(ebuf_d, spm_t.at[sl])
    zcp.wait()
    plsc.subcore_barrier()

    for j in range(6):
        pltpu.sync_copy(spm_t, t_loc)

        def _spmv16(s, d):
            v = plsc.load_gather(t_loc, [s])
            plsc.addupdate_scatter(u_loc, [d >> 7, d & 127], v)
        _edge_loop(False, _spmv16)
        zcp = _publish()
        plsc.subcore_barrier()

        pltpu.sync_copy(spm_u.at[rsl], ubuf)
        _reset_spm_u_rows()
        @plsc.parallel_loop(0, NVR, unroll=4)
        def _(k):
            r = k // 8
            c = lax.rem(k, 8)
            off = k * 16
            acc = ubuf[r, pl.ds(c * 16, 16)]
            t = t_loc[pl.ds(wid * NS + off, 16)]
            dv = ebuf_d[pl.ds(off, 16)]
            a = dv * (acc + t)
            ebuf_a[pl.ds(off, 16)] = a
            ebuf_t[pl.ds(off, 16)] = dv * a
        pltpu.sync_copy(ebuf_a, a_out.at[j, sl])
        pltpu.sync_copy(ebuf_t, spm_t.at[sl])
        zcp.wait()
        plsc.subcore_barrier()


_sc_spmv = functools.partial(
    pl.kernel,
    out_type=jax.ShapeDtypeStruct((6, NP), jnp.float32),
    mesh=plsc.VectorSubcoreMesh(
        core_axis_name="c", subcore_axis_name="s", num_cores=1, num_subcores=16
    ),
    compiler_params=pltpu.CompilerParams(needs_layout_passes=False),
    scratch_types=[
        pltpu.VMEM((NP,), jnp.float32),
        pltpu.VMEM((NROW, 128), jnp.float32),
        pltpu.VMEM((CH,), jnp.int32),
        pltpu.VMEM((CH,), jnp.int32),
        pltpu.VMEM((CH,), jnp.int32),
        pltpu.VMEM((CH,), jnp.int32),
        pltpu.VMEM((CH,), jnp.int32),
        pltpu.VMEM((CH,), jnp.int32),
        pltpu.VMEM((RPT, 128), jnp.float32),
        pltpu.VMEM((ZR, 128), jnp.float32),
        pltpu.VMEM((NPUB, PUB), jnp.int32),
        pltpu.VMEM((NS,), jnp.float32),
        pltpu.VMEM((NS,), jnp.float32),
        pltpu.VMEM((NS,), jnp.float32),
        pltpu.VMEM_SHARED((NP,), jnp.float32),
        pltpu.VMEM_SHARED((NROW, 128), jnp.float32),
        pltpu.SemaphoreType.DMA,
        pltpu.SemaphoreType.DMA,
        pltpu.SemaphoreType.DMA,
        pltpu.SemaphoreType.DMA,
    ],
)(_sc_body)


def _tc_body(a_ref, W1, b1, W2, b2, W3, b3, W4, b4, W5, b5, W6, b6, out_ref):
    rs = [b1[...], W1[...]]
    for Wr, br in ((W2, b2), (W3, b3), (W4, b4), (W5, b5), (W6, b6)):
        W = Wr[...]
        rs = [br[...]] + [jnp.dot(r, W) for r in rs]
    y = jnp.zeros((1, NP), jnp.float32) + rs[0]
    for j in range(1, 7):
        y = y + rs[j] * a_ref[j - 1:j, :]
    col = lax.broadcasted_iota(jnp.int32, (1, NP), 1)
    valid = col < N
    ymin = jnp.min(jnp.where(valid, y, jnp.inf), keepdims=True)
    ymax = jnp.max(jnp.where(valid, y, -jnp.inf), keepdims=True)
    out_ref[...] = (y - ymin) / (ymax - ymin + 1e-15)


def kernel(edge_index, W1, b1, W2, b2, W3, b3, W4, b4, W5, b5, W6, b6):
    ei_flat = edge_index.astype(jnp.int32).reshape(2 * E)
    a_stack = _sc_spmv(ei_flat, jnp.zeros((NROW, 128), jnp.float32))
    y = pl.pallas_call(
        _tc_body,
        out_shape=jax.ShapeDtypeStruct((1, NP), jnp.float32),
    )(a_stack, W1, b1.reshape(1, -1), W2, b2.reshape(1, -1),
      W3, b3.reshape(1, -1), W4, b4.reshape(1, -1),
      W5, b5.reshape(1, -1), W6, b6.reshape(1, -1))
    return y[0, :N].reshape(N, 1)


# speedup vs baseline: 1.2687x; 1.2687x over previous
import functools

import jax
import jax.numpy as jnp
from jax import lax
from jax.experimental import pallas as pl
from jax.experimental.pallas import tpu as pltpu
from jax.experimental.pallas import tpu_sc as plsc

N = 50000
E = 1600000
NTILES = 16
NP = 51200
NROW = NP // 128
NS = NP // NTILES
NVR = NS // 16
RPT = NROW // NTILES
ZR = 5
EPT = E // NTILES
CH = 800
NCH = EPT // CH
CHV = CH // 16
NBANK = 4
PUB = 80
NPUB = NROW // PUB


def _rsqrt16(x):
    xi = plsc.bitcast(x, jnp.int32)
    yi = jnp.int32(0x5F3759DF) - (xi >> 1)
    y = plsc.bitcast(yi, jnp.float32)
    for _ in range(3):
        y = y * (1.5 - 0.5 * x * y * y)
    return y


def _sc_body(ei_hbm, zeros_hbm, a_out,
             t_loc, u_loc, sb0, db0, sb1, db1, sb2, db2, sb3, db3,
             ubuf, zbuf, idxq, ebuf_d, ebuf_a, spm_t, spm_u,
             sem0, sem1, sem2, sem3, semz):
    wid = lax.axis_index("s")
    sl = pl.ds(wid * NS, NS)
    rsl = pl.ds(wid * RPT, RPT)
    ebase = wid * EPT
    zero16 = jnp.zeros((16,), jnp.float32)
    ones16 = jnp.ones((16,), jnp.float32)
    iota16 = lax.iota(jnp.int32, 16)

    def _z(k, _):
        r = k // 8
        c = lax.rem(k, 8)
        zbuf[r, pl.ds(c * 16, 16)] = zero16
        return 0
    lax.fori_loop(0, ZR * 8, _z, 0)
    for q in range(NPUB):
        def _iq(k, _):
            idxq[q, pl.ds(k * 16, 16)] = iota16 + (q * PUB + k * 16)
            return 0
        lax.fori_loop(0, PUB // 16, _iq, 0)

    def _reset_spm_u_rows():
        cps = [pltpu.async_copy(zbuf, spm_u.at[pl.ds(wid * RPT + g * ZR, ZR)], sem0)
               for g in range(RPT // ZR)]
        for cp in cps:
            cp.wait()

    _reset_spm_u_rows()
    zcp = pltpu.async_copy(zeros_hbm, u_loc, semz)
    zcp.wait()
    plsc.subcore_barrier()

    banks = ((sb0, db0, sem0), (sb1, db1, sem1),
             (sb2, db2, sem2), (sb3, db3, sem3))

    def _edge_loop(deg_only, process16):
        def _fetch(c, bank):
            sb, db, sem = banks[bank]
            if not deg_only:
                pltpu.async_copy(ei_hbm.at[pl.ds(ebase + c * CH, CH)], sb, sem)
            pltpu.async_copy(ei_hbm.at[pl.ds(E + ebase + c * CH, CH)], db, sem)

        for b in range(NBANK - 1):
            _fetch(b, b)

        def _chunk(c, _):
            def _go(bank):
                sb, db, sem = banks[bank]
                for _ in range(1 if deg_only else 2):
                    pltpu.make_async_copy(
                        ei_hbm.at[pl.ds(0, CH)], db, sem).wait()
                @pl.when(c + (NBANK - 1) < NCH)
                def _():
                    _fetch(c + (NBANK - 1), (bank + NBANK - 1) % NBANK)
                @plsc.parallel_loop(0, CHV, unroll=10)
                def _(i):
                    process16(sb[pl.ds(i * 16, 16)], db[pl.ds(i * 16, 16)])
            for b in range(NBANK):
                @pl.when(lax.rem(c, NBANK) == b)
                def _(b=b):
                    _go(b)
            return 0
        lax.fori_loop(0, NCH, _chunk, 0)

    def _publish():
        cps = [pltpu.async_copy(u_loc.at[pl.ds(q * PUB, PUB)],
                                spm_u.at[idxq.at[q]], sem0, add=True)
               for q in range(NPUB)]
        for cp in cps:
            cp.wait()
        return pltpu.async_copy(zeros_hbm, u_loc, semz)

    def _deg16(s, d):
        del s
        plsc.addupdate_scatter(u_loc, [d >> 7, d & 127], ones16)
    _edge_loop(True, _deg16)
    zcp = _publish()
    plsc.subcore_barrier()

    pltpu.sync_copy(spm_u.at[rsl], ubuf)
    _reset_spm_u_rows()
    @plsc.parallel_loop(0, NVR, unroll=4)
    def _(k):
        r = k // 8
        c = lax.rem(k, 8)
        acc = ubuf[r, pl.ds(c * 16, 16)]
        dv = _rsqrt16(acc + 1.0)
        ebuf_d[pl.ds(k * 16, 16)] = dv
        ubuf[r, pl.ds(c * 16, 16)] = dv
    pltpu.sync_copy(ubuf, spm_t.at[rsl])
    zcp.wait()
    plsc.subcore_barrier()

    for j in range(6):
        pltpu.sync_copy(spm_t, t_loc)

        def _spmv16(s, d):
            v = plsc.load_gather(t_loc, [s >> 7, s & 127])
            plsc.addupdate_scatter(u_loc, [d >> 7, d & 127], v)
        _edge_loop(False, _spmv16)
        zcp = _publish()
        plsc.subcore_barrier()

        pltpu.sync_copy(spm_u.at[rsl], ubuf)
        _reset_spm_u_rows()
        @plsc.parallel_loop(0, NVR, unroll=4)
        def _(k):
            r = k // 8
            c = lax.rem(k, 8)
            off = k * 16
            acc = ubuf[r, pl.ds(c * 16, 16)]
            t = t_loc[wid * RPT + r, pl.ds(c * 16, 16)]
            dv = ebuf_d[pl.ds(off, 16)]
            a = dv * (acc + t)
            ebuf_a[pl.ds(off, 16)] = a
            ubuf[r, pl.ds(c * 16, 16)] = dv * a
        pltpu.sync_copy(ebuf_a, a_out.at[j, sl])
        pltpu.sync_copy(ubuf, spm_t.at[rsl])
        zcp.wait()
        plsc.subcore_barrier()


_sc_spmv = functools.partial(
    pl.kernel,
    out_type=jax.ShapeDtypeStruct((6, NP), jnp.float32),
    mesh=plsc.VectorSubcoreMesh(
        core_axis_name="c", subcore_axis_name="s", num_cores=1, num_subcores=16
    ),
    compiler_params=pltpu.CompilerParams(needs_layout_passes=False),
    scratch_types=[
        pltpu.VMEM((NROW, 128), jnp.float32),
        pltpu.VMEM((NROW, 128), jnp.float32),
        pltpu.VMEM((CH,), jnp.int32),
        pltpu.VMEM((CH,), jnp.int32),
        pltpu.VMEM((CH,), jnp.int32),
        pltpu.VMEM((CH,), jnp.int32),
        pltpu.VMEM((CH,), jnp.int32),
        pltpu.VMEM((CH,), jnp.int32),
        pltpu.VMEM((CH,), jnp.int32),
        pltpu.VMEM((CH,), jnp.int32),
        pltpu.VMEM((RPT, 128), jnp.float32),
        pltpu.VMEM((ZR, 128), jnp.float32),
        pltpu.VMEM((NPUB, PUB), jnp.int32),
        pltpu.VMEM((NS,), jnp.float32),
        pltpu.VMEM((NS,), jnp.float32),
        pltpu.VMEM_SHARED((NROW, 128), jnp.float32),
        pltpu.VMEM_SHARED((NROW, 128), jnp.float32),
        pltpu.SemaphoreType.DMA,
        pltpu.SemaphoreType.DMA,
        pltpu.SemaphoreType.DMA,
        pltpu.SemaphoreType.DMA,
        pltpu.SemaphoreType.DMA,
    ],
)(_sc_body)


def _tc_body(a_ref, W1, b1, W2, b2, W3, b3, W4, b4, W5, b5, W6, b6, out_ref):
    rs = [b1[...], W1[...]]
    for Wr, br in ((W2, b2), (W3, b3), (W4, b4), (W5, b5), (W6, b6)):
        W = Wr[...]
        rs = [br[...]] + [jnp.dot(r, W) for r in rs]
    y = jnp.zeros((1, NP), jnp.float32) + rs[0]
    for j in range(1, 7):
        y = y + rs[j] * a_ref[j - 1:j, :]
    col = lax.broadcasted_iota(jnp.int32, (1, NP), 1)
    valid = col < N
    ymin = jnp.min(jnp.where(valid, y, jnp.inf), keepdims=True)
    ymax = jnp.max(jnp.where(valid, y, -jnp.inf), keepdims=True)
    out_ref[...] = (y - ymin) / (ymax - ymin + 1e-15)


def kernel(edge_index, W1, b1, W2, b2, W3, b3, W4, b4, W5, b5, W6, b6):
    ei_flat = edge_index.astype(jnp.int32).reshape(2 * E)
    a_stack = _sc_spmv(ei_flat, jnp.zeros((NROW, 128), jnp.float32))
    y = pl.pallas_call(
        _tc_body,
        out_shape=jax.ShapeDtypeStruct((1, NP), jnp.float32),
    )(a_stack, W1, b1.reshape(1, -1), W2, b2.reshape(1, -1),
      W3, b3.reshape(1, -1), W4, b4.reshape(1, -1),
      W5, b5.reshape(1, -1), W6, b6.reshape(1, -1))
    return y[0, :N].reshape(N, 1)
